# baseline (speedup 1.0000x reference)
import jax
import jax.numpy as jnp
from jax import lax
from jax.experimental import pallas as pl
from jax.experimental.pallas import tpu as pltpu
from jax.experimental.pallas import tpu_sc as plsc

N_USERS = 50000
N_NODES = 100000
D = 64
E = 1250000

NC = 2
NS = 16
L = 16

P = 25600
N_PAD = 4 * P
B = 128
E_PAD = ((E + NS * B - 1) // (NS * B)) * (NS * B)
TE = E_PAD // NS
ROWS_TILE = P // NS
ZROWS = 160

SCC = 6016
NSC = TE // SCC
KR = SCC // B
QTILE = TE
Q = 4 * NS * QTILE
TRASH = Q
DUMMY = P

_MESH = plsc.VectorSubcoreMesh(core_axis_name="c", subcore_axis_name="s",
                               num_cores=NC, num_subcores=NS)
_PARAMS = pltpu.CompilerParams(use_tc_tiling_on_sc=False)


def _bucket_body(row_hbm, col_hbm, val_hbm,
                 colq_hbm, valq_hbm, idxq_hbm, cnt_hbm,
                 rowbuf, colbuf, valbuf,
                 tgt0, tgt1, lrb0, lrb1, cntbuf, sem):
    c = lax.axis_index("c")
    s = lax.axis_index("s")
    lanes = lax.iota(jnp.int32, L)
    last = jnp.full((L,), L - 1, jnp.int32)

    def psum16(mi):
        p = mi
        for sh in (1, 2, 4, 8):
            idx = jnp.maximum(lanes - sh, 0)
            shifted = p.at[idx].get(mode="promise_in_bounds")
            p = p + jnp.where(lanes >= sh, shifted, jnp.int32(0))
        return p

    def batch(i, carry):
        o0, o1 = carry
        eb = s * TE + i * B
        pltpu.sync_copy(row_hbm.at[pl.ds(eb, B)], rowbuf)
        pltpu.sync_copy(col_hbm.at[pl.ds(eb, B)], colbuf)
        pltpu.sync_copy(val_hbm.at[pl.ds(eb, B)], valbuf)
        for j in range(B // L):
            r = rowbuf[pl.ds(j * L, L)]
            for b in range(2):
                tgt, lrb = (tgt0, lrb0) if b == 0 else (tgt1, lrb1)
                o = o0 if b == 0 else o1
                qbase = ((2 * c + b) * NS + s) * QTILE
                lr = r - (2 * c + b) * P
                m = (lr >= 0) & (lr < P)
                mi = jnp.where(m, jnp.int32(1), jnp.int32(0))
                pos = psum16(mi)
                t = jnp.where(m, (qbase - 1) + (o + pos),
                              jnp.full((L,), TRASH, jnp.int32))
                tgt[pl.ds(j * L, L)] = t
                lrb[pl.ds(j * L, L)] = jnp.where(m, lr, DUMMY)
                o = o + pos.at[last].get(mode="promise_in_bounds")
                if b == 0:
                    o0 = o
                else:
                    o1 = o
        descs = [
            pltpu.async_copy(colbuf, colq_hbm.at[tgt0], sem),
            pltpu.async_copy(colbuf, colq_hbm.at[tgt1], sem),
            pltpu.async_copy(valbuf, valq_hbm.at[tgt0], sem),
            pltpu.async_copy(valbuf, valq_hbm.at[tgt1], sem),
            pltpu.async_copy(lrb0, idxq_hbm.at[tgt0], sem),
            pltpu.async_copy(lrb1, idxq_hbm.at[tgt1], sem),
        ]
        for d in descs:
            d.wait()
        return (o0, o1)

    z = jnp.zeros((L,), jnp.int32)
    o0, o1 = lax.fori_loop(0, TE // B, batch, (z, z))

    cntbuf[pl.ds(0, L)] = o0
    pltpu.sync_copy(cntbuf, cnt_hbm.at[pl.ds(((2 * c) * NS + s) * L, L)])
    cntbuf[pl.ds(0, L)] = o1
    pltpu.sync_copy(cntbuf, cnt_hbm.at[pl.ds(((2 * c + 1) * NS + s) * L, L)])


_bucket = pl.kernel(
    _bucket_body,
    out_type=[
        jax.ShapeDtypeStruct((Q + 64,), jnp.int32),
        jax.ShapeDtypeStruct((Q + 64,), jnp.float32),
        jax.ShapeDtypeStruct((Q + 64,), jnp.int32),
        jax.ShapeDtypeStruct((4 * NS * L,), jnp.int32),
    ],
    mesh=_MESH,
    scratch_types=[
        pltpu.VMEM((B,), jnp.int32),
        pltpu.VMEM((B,), jnp.int32),
        pltpu.VMEM((B,), jnp.float32),
        pltpu.VMEM((B,), jnp.int32),
        pltpu.VMEM((B,), jnp.int32),
        pltpu.VMEM((B,), jnp.int32),
        pltpu.VMEM((B,), jnp.int32),
        pltpu.VMEM((L,), jnp.int32),
        pltpu.SemaphoreType.DMA,
    ],
    compiler_params=_PARAMS,
)


def _hop_body(ego_hbm, colq_hbm, valq_hbm, idxq_hbm, cnt_hbm, out_hbm,
              colbuf, valbuf, idxbuf, gbuf, zbuf, cntv, acc, sem):
    c = lax.axis_index("c")
    s = lax.axis_index("s")
    lanes = lax.iota(jnp.int32, L)

    zeros = jnp.zeros((L,), jnp.float32)
    for i in range(ZROWS):
        for j in range(D // L):
            zbuf[i, pl.ds(j * L, L)] = zeros

    for p in range(2):
        base = (2 * c + p) * P
        qbase = ((2 * c + p) * NS + s) * QTILE

        for i in range(ROWS_TILE // ZROWS):
            pltpu.sync_copy(zbuf, acc.at[pl.ds(s * ROWS_TILE + i * ZROWS, ZROWS)])
        plsc.subcore_barrier()

        pltpu.sync_copy(cnt_hbm.at[pl.ds(((2 * c + p) * NS + s) * L, L)], cntv)
        cnt = cntv[pl.ds(0, L)]
        nb = (cnt[0] + (B - 1)) // B

        def batch_body(i, carry):
            qb = qbase + i * B
            _do_batch(i, qb)
            return carry

        def _do_batch(i, qb):
            pltpu.sync_copy(colq_hbm.at[pl.ds(qb, B)], colbuf)
            pltpu.sync_copy(valq_hbm.at[pl.ds(qb, B)], valbuf)
            pltpu.sync_copy(idxq_hbm.at[pl.ds(qb, B)], idxbuf)
            for g in range(B // L):
                gpos = lanes + (i * B + g * L)
                ok = gpos < cnt
                cl = colbuf[pl.ds(g * L, L)]
                colbuf[pl.ds(g * L, L)] = jnp.where(ok, cl, 0)
                li = idxbuf[pl.ds(g * L, L)]
                idxbuf[pl.ds(g * L, L)] = jnp.where(ok, li, DUMMY)
            pltpu.async_copy(ego_hbm.at[colbuf], gbuf, sem).wait()
            for g in range(B // L):
                vals_g = valbuf[pl.ds(g * L, L)]
                for jj in range(L):
                    j = g * L + jj
                    vj = vals_g.at[jnp.full((L,), jj, jnp.int32)].get(
                        mode="promise_in_bounds")
                    for f in range(D // L):
                        gbuf[j, pl.ds(f * L, L)] = gbuf[j, pl.ds(f * L, L)] * vj
            pltpu.sync_copy(gbuf, acc.at[idxbuf], add=True)

        lax.fori_loop(0, nb, batch_body, 0)
        plsc.subcore_barrier()
        pltpu.sync_copy(acc.at[pl.ds(s * ROWS_TILE, ROWS_TILE)],
                        out_hbm.at[pl.ds(base + s * ROWS_TILE, ROWS_TILE)])
        plsc.subcore_barrier()


_hop = pl.kernel(
    _hop_body,
    out_type=jax.ShapeDtypeStruct((N_PAD, D), jnp.float32),
    mesh=_MESH,
    scratch_types=[
        pltpu.VMEM((B,), jnp.int32),
        pltpu.VMEM((B,), jnp.float32),
        pltpu.VMEM((B,), jnp.int32),
        pltpu.VMEM((B, D), jnp.float32),
        pltpu.VMEM((ZROWS, D), jnp.float32),
        pltpu.VMEM((L,), jnp.int32),
        pltpu.VMEM_SHARED((P + 8, D), jnp.float32),
        pltpu.SemaphoreType.DMA,
    ],
    compiler_params=_PARAMS,
)


def kernel(user_embed, item_embed, adj_row, adj_col, adj_val):
    ego0 = jnp.concatenate([user_embed, item_embed], axis=0)
    ego0 = jnp.pad(ego0, ((0, N_PAD - N_NODES), (0, 0)))
    row = jnp.pad(adj_row.astype(jnp.int32), (0, E_PAD - E),
                  constant_values=jnp.int32(1 << 20))
    col = jnp.pad(adj_col.astype(jnp.int32), (0, E_PAD - E))
    val = jnp.pad(adj_val, (0, E_PAD - E))

    colq, valq, idxq, cnts = _bucket(row, col, val)
    e1 = _hop(ego0, colq, valq, idxq, cnts)
    e2 = _hop(e1, colq, valq, idxq, cnts)
    e3 = _hop(e2, colq, valq, idxq, cnts)

    mean = (e1 + e2 + e3) * jnp.float32(1.0 / 3.0)
    user_all = mean[:N_USERS]
    item_all = mean[N_USERS:N_NODES]
    user_layer = e1[:N_USERS]
    item_layer = e1[N_USERS:N_NODES]
    return (user_all, item_all, user_layer, item_layer)


# speedup vs baseline: 132.0738x; 132.0738x over previous
import jax
import jax.numpy as jnp
from jax import lax
from jax.experimental import pallas as pl
from jax.experimental.pallas import tpu as pltpu
from jax.experimental.pallas import tpu_sc as plsc

N_USERS = 50000
N_NODES = 100000
D = 64
E = 1250000

NC = 2
NS = 16
L = 16

P = 25600
N_PAD = 4 * P
B = 128
E_PAD = ((E + NS * B - 1) // (NS * B)) * (NS * B)
TE = E_PAD // NS
ROWS_TILE = P // NS
ZROWS = 160

SCC = 6016
NSC = TE // SCC
KR = SCC // B
CH = 1024
QTILE = ((TE + CH - 1) // CH) * CH
Q = 4 * NS * QTILE
TRASH = Q
DUMMY = P

_MESH = plsc.VectorSubcoreMesh(core_axis_name="c", subcore_axis_name="s",
                               num_cores=NC, num_subcores=NS)
_PARAMS = pltpu.CompilerParams(use_tc_tiling_on_sc=False)


def _bucket_body(row_hbm, col_hbm, val_hbm,
                 colq_hbm, valq_hbm, idxq_hbm, cnt_hbm,
                 rowbuf, colbuf, valbuf,
                 qcol0, qval0, qidx0, qcol1, qval1, qidx1, cntbuf):
    c = lax.axis_index("c")
    s = lax.axis_index("s")
    lanes = lax.iota(jnp.int32, L)

    def psum16(mi):
        p = mi
        for sh in (1, 2, 4, 8):
            idx = jnp.maximum(lanes - sh, 0)
            shifted = p.at[idx].get(mode="promise_in_bounds")
            p = p + jnp.where(lanes >= sh, shifted, jnp.int32(0))
        return p

    qbufs = ((qcol0, qval0, qidx0), (qcol1, qval1, qidx1))

    def flush(b, nch):
        qcol, qval, qidx = qbufs[b]
        qbase = ((2 * c + b) * NS + s) * QTILE + nch * CH
        pltpu.sync_copy(qcol.at[pl.ds(0, CH)], colq_hbm.at[pl.ds(qbase, CH)])
        pltpu.sync_copy(qval.at[pl.ds(0, CH)], valq_hbm.at[pl.ds(qbase, CH)])
        pltpu.sync_copy(qidx.at[pl.ds(0, CH)], idxq_hbm.at[pl.ds(qbase, CH)])

    def batch(i, carry):
        o0, o1, n0, n1 = carry
        eb = s * TE + i * B
        pltpu.sync_copy(row_hbm.at[pl.ds(eb, B)], rowbuf)
        pltpu.sync_copy(col_hbm.at[pl.ds(eb, B)], colbuf)
        pltpu.sync_copy(val_hbm.at[pl.ds(eb, B)], valbuf)
        for j in range(B // L):
            r = rowbuf[pl.ds(j * L, L)]
            cl = colbuf[pl.ds(j * L, L)]
            vv = valbuf[pl.ds(j * L, L)]
            for b in range(2):
                qcol, qval, qidx = qbufs[b]
                o = o0 if b == 0 else o1
                lr = r - (2 * c + b) * P
                m = (lr >= 0) & (lr < P)
                mi = jnp.where(m, jnp.int32(1), jnp.int32(0))
                pos = psum16(mi)
                posm = jnp.where(m, pos, jnp.int32(0))
                srcv = lanes
                for l in range(L):
                    pm = posm.at[jnp.full((L,), l, jnp.int32)].get(
                        mode="promise_in_bounds")
                    srcv = jnp.where(lanes + 1 == pm, jnp.int32(l), srcv)
                qcol[pl.ds(o, L)] = cl.at[srcv].get(mode="promise_in_bounds")
                qval[pl.ds(o, L)] = vv.at[srcv].get(mode="promise_in_bounds")
                lrd = jnp.where(m, lr, DUMMY)
                qidx[pl.ds(o, L)] = lrd.at[srcv].get(mode="promise_in_bounds")
                o = o + pos[L - 1]
                if b == 0:
                    o0 = o
                else:
                    o1 = o
        for b in range(2):
            o, nch = (o0, n0) if b == 0 else (o1, n1)
            qcol, qval, qidx = qbufs[b]

            @pl.when(o >= CH)
            def _():
                flush(b, nch)
                for k in range(B // L):
                    rc = qcol[pl.ds(CH + k * L, L)]
                    rv = qval[pl.ds(CH + k * L, L)]
                    ri = qidx[pl.ds(CH + k * L, L)]
                    qcol[pl.ds(k * L, L)] = rc
                    qval[pl.ds(k * L, L)] = rv
                    qidx[pl.ds(k * L, L)] = ri

            nch = nch + jnp.where(o >= CH, jnp.int32(1), jnp.int32(0))
            o = jnp.where(o >= CH, o - CH, o)
            if b == 0:
                o0, n0 = o, nch
            else:
                o1, n1 = o, nch
        return (o0, o1, n0, n1)

    z = jnp.int32(0)
    o0, o1, n0, n1 = lax.fori_loop(0, TE // B, batch, (z, z, z, z))

    for b in range(2):
        o, nch = (o0, n0) if b == 0 else (o1, n1)

        @pl.when(o > 0)
        def _():
            flush(b, nch)
        cntbuf[pl.ds(0, L)] = jnp.full((L,), nch * CH + o, jnp.int32)
        pltpu.sync_copy(cntbuf,
                        cnt_hbm.at[pl.ds(((2 * c + b) * NS + s) * L, L)])


_bucket = pl.kernel(
    _bucket_body,
    out_type=[
        jax.ShapeDtypeStruct((Q + 64,), jnp.int32),
        jax.ShapeDtypeStruct((Q + 64,), jnp.float32),
        jax.ShapeDtypeStruct((Q + 64,), jnp.int32),
        jax.ShapeDtypeStruct((4 * NS * L,), jnp.int32),
    ],
    mesh=_MESH,
    scratch_types=[
        pltpu.VMEM((B,), jnp.int32),
        pltpu.VMEM((B,), jnp.int32),
        pltpu.VMEM((B,), jnp.float32),
        pltpu.VMEM((CH + B,), jnp.int32),
        pltpu.VMEM((CH + B,), jnp.float32),
        pltpu.VMEM((CH + B,), jnp.int32),
        pltpu.VMEM((CH + B,), jnp.int32),
        pltpu.VMEM((CH + B,), jnp.float32),
        pltpu.VMEM((CH + B,), jnp.int32),
        pltpu.VMEM((L,), jnp.int32),
    ],
    compiler_params=_PARAMS,
)


def _hop_body(ego_hbm, colq_hbm, valq_hbm, idxq_hbm, cnt_hbm, out_hbm,
              colbuf, valbuf, idxbuf, gbuf, zbuf, cntv, acc, sem):
    c = lax.axis_index("c")
    s = lax.axis_index("s")
    lanes = lax.iota(jnp.int32, L)

    zeros = jnp.zeros((L,), jnp.float32)
    for i in range(ZROWS):
        for j in range(D // L):
            zbuf[i, pl.ds(j * L, L)] = zeros

    for p in range(2):
        base = (2 * c + p) * P
        qbase = ((2 * c + p) * NS + s) * QTILE

        for i in range(ROWS_TILE // ZROWS):
            pltpu.sync_copy(zbuf, acc.at[pl.ds(s * ROWS_TILE + i * ZROWS, ZROWS)])
        plsc.subcore_barrier()

        pltpu.sync_copy(cnt_hbm.at[pl.ds(((2 * c + p) * NS + s) * L, L)], cntv)
        cnt = cntv[pl.ds(0, L)]
        nb = (cnt[0] + (B - 1)) // B

        def batch_body(i, carry):
            qb = qbase + i * B
            _do_batch(i, qb)
            return carry

        def _do_batch(i, qb):
            pltpu.sync_copy(colq_hbm.at[pl.ds(qb, B)], colbuf)
            pltpu.sync_copy(valq_hbm.at[pl.ds(qb, B)], valbuf)
            pltpu.sync_copy(idxq_hbm.at[pl.ds(qb, B)], idxbuf)
            for g in range(B // L):
                gpos = lanes + (i * B + g * L)
                ok = gpos < cnt
                cl = colbuf[pl.ds(g * L, L)]
                colbuf[pl.ds(g * L, L)] = jnp.where(ok, cl, 0)
                li = idxbuf[pl.ds(g * L, L)]
                idxbuf[pl.ds(g * L, L)] = jnp.where(ok, li, DUMMY)
            pltpu.async_copy(ego_hbm.at[colbuf], gbuf, sem).wait()
            for g in range(B // L):
                vals_g = valbuf[pl.ds(g * L, L)]
                for jj in range(L):
                    j = g * L + jj
                    vj = vals_g.at[jnp.full((L,), jj, jnp.int32)].get(
                        mode="promise_in_bounds")
                    for f in range(D // L):
                        gbuf[j, pl.ds(f * L, L)] = gbuf[j, pl.ds(f * L, L)] * vj
            pltpu.sync_copy(gbuf, acc.at[idxbuf], add=True)

        lax.fori_loop(0, nb, batch_body, 0)
        plsc.subcore_barrier()
        pltpu.sync_copy(acc.at[pl.ds(s * ROWS_TILE, ROWS_TILE)],
                        out_hbm.at[pl.ds(base + s * ROWS_TILE, ROWS_TILE)])
        plsc.subcore_barrier()


_hop = pl.kernel(
    _hop_body,
    out_type=jax.ShapeDtypeStruct((N_PAD, D), jnp.float32),
    mesh=_MESH,
    scratch_types=[
        pltpu.VMEM((B,), jnp.int32),
        pltpu.VMEM((B,), jnp.float32),
        pltpu.VMEM((B,), jnp.int32),
        pltpu.VMEM((B, D), jnp.float32),
        pltpu.VMEM((ZROWS, D), jnp.float32),
        pltpu.VMEM((L,), jnp.int32),
        pltpu.VMEM_SHARED((P + 8, D), jnp.float32),
        pltpu.SemaphoreType.DMA,
    ],
    compiler_params=_PARAMS,
)


def kernel(user_embed, item_embed, adj_row, adj_col, adj_val):
    ego0 = jnp.concatenate([user_embed, item_embed], axis=0)
    ego0 = jnp.pad(ego0, ((0, N_PAD - N_NODES), (0, 0)))
    row = jnp.pad(adj_row.astype(jnp.int32), (0, E_PAD - E),
                  constant_values=jnp.int32(1 << 20))
    col = jnp.pad(adj_col.astype(jnp.int32), (0, E_PAD - E))
    val = jnp.pad(adj_val, (0, E_PAD - E))

    colq, valq, idxq, cnts = _bucket(row, col, val)
    e1 = _hop(ego0, colq, valq, idxq, cnts)
    e2 = _hop(e1, colq, valq, idxq, cnts)
    e3 = _hop(e2, colq, valq, idxq, cnts)

    mean = (e1 + e2 + e3) * jnp.float32(1.0 / 3.0)
    user_all = mean[:N_USERS]
    item_all = mean[N_USERS:N_NODES]
    user_layer = e1[:N_USERS]
    item_layer = e1[N_USERS:N_NODES]
    return (user_all, item_all, user_layer, item_layer)


# speedup vs baseline: 208.5478x; 1.5790x over previous
import jax
import jax.numpy as jnp
from jax import lax
from jax.experimental import pallas as pl
from jax.experimental.pallas import tpu as pltpu
from jax.experimental.pallas import tpu_sc as plsc

N_USERS = 50000
N_NODES = 100000
D = 64
E = 1250000

NC = 2
NS = 16
L = 16

P = 25600
N_PAD = 4 * P
B = 128
E_PAD = ((E + NS * B - 1) // (NS * B)) * (NS * B)
TE = E_PAD // NS
ROWS_TILE = P // NS
ZROWS = 160

SCC = 6016
NSC = TE // SCC
KR = SCC // B
CH = 1024
QTILE = ((TE + CH - 1) // CH) * CH
Q = 4 * NS * QTILE
TRASH = Q
DUMMY = P

_MESH = plsc.VectorSubcoreMesh(core_axis_name="c", subcore_axis_name="s",
                               num_cores=NC, num_subcores=NS)
_PARAMS = pltpu.CompilerParams(use_tc_tiling_on_sc=False)


def _bucket_body(row_hbm, col_hbm, val_hbm,
                 colq_hbm, valq_hbm, idxq_hbm, cnt_hbm,
                 rowbuf, colbuf, valbuf,
                 qcol0, qval0, qidx0, qcol1, qval1, qidx1, cntbuf):
    c = lax.axis_index("c")
    s = lax.axis_index("s")
    lanes = lax.iota(jnp.int32, L)

    def psum16(mi):
        p = mi
        for sh in (1, 2, 4, 8):
            idx = jnp.maximum(lanes - sh, 0)
            shifted = p.at[idx].get(mode="promise_in_bounds")
            p = p + jnp.where(lanes >= sh, shifted, jnp.int32(0))
        return p

    qbufs = ((qcol0, qval0, qidx0), (qcol1, qval1, qidx1))

    def flush(b, nch):
        qcol, qval, qidx = qbufs[b]
        qbase = ((2 * c + b) * NS + s) * QTILE + nch * CH
        pltpu.sync_copy(qcol.at[pl.ds(0, CH)], colq_hbm.at[pl.ds(qbase, CH)])
        pltpu.sync_copy(qval.at[pl.ds(0, CH)], valq_hbm.at[pl.ds(qbase, CH)])
        pltpu.sync_copy(qidx.at[pl.ds(0, CH)], idxq_hbm.at[pl.ds(qbase, CH)])

    def batch(i, carry):
        o0, o1, n0, n1 = carry
        eb = s * TE + i * B
        pltpu.sync_copy(row_hbm.at[pl.ds(eb, B)], rowbuf)
        pltpu.sync_copy(col_hbm.at[pl.ds(eb, B)], colbuf)
        pltpu.sync_copy(val_hbm.at[pl.ds(eb, B)], valbuf)
        for j in range(B // L):
            r = rowbuf[pl.ds(j * L, L)]
            cl = colbuf[pl.ds(j * L, L)]
            vv = valbuf[pl.ds(j * L, L)]
            for b in range(2):
                qcol, qval, qidx = qbufs[b]
                o = o0 if b == 0 else o1
                lr = r - (2 * c + b) * P
                m = (lr >= 0) & (lr < P)
                mi = jnp.where(m, jnp.int32(1), jnp.int32(0))
                pos = psum16(mi)
                posm = jnp.where(m, pos, jnp.int32(0))
                srcv = lanes
                for l in range(L):
                    pm = posm.at[jnp.full((L,), l, jnp.int32)].get(
                        mode="promise_in_bounds")
                    srcv = jnp.where(lanes + 1 == pm, jnp.int32(l), srcv)
                qcol[pl.ds(o, L)] = cl.at[srcv].get(mode="promise_in_bounds")
                qval[pl.ds(o, L)] = vv.at[srcv].get(mode="promise_in_bounds")
                lrd = jnp.where(m, lr, DUMMY)
                qidx[pl.ds(o, L)] = lrd.at[srcv].get(mode="promise_in_bounds")
                o = o + pos[L - 1]
                if b == 0:
                    o0 = o
                else:
                    o1 = o
        for b in range(2):
            o, nch = (o0, n0) if b == 0 else (o1, n1)
            qcol, qval, qidx = qbufs[b]

            @pl.when(o >= CH)
            def _():
                flush(b, nch)
                for k in range(B // L):
                    rc = qcol[pl.ds(CH + k * L, L)]
                    rv = qval[pl.ds(CH + k * L, L)]
                    ri = qidx[pl.ds(CH + k * L, L)]
                    qcol[pl.ds(k * L, L)] = rc
                    qval[pl.ds(k * L, L)] = rv
                    qidx[pl.ds(k * L, L)] = ri

            nch = nch + jnp.where(o >= CH, jnp.int32(1), jnp.int32(0))
            o = jnp.where(o >= CH, o - CH, o)
            if b == 0:
                o0, n0 = o, nch
            else:
                o1, n1 = o, nch
        return (o0, o1, n0, n1)

    z = jnp.int32(0)
    o0, o1, n0, n1 = lax.fori_loop(0, TE // B, batch, (z, z, z, z))

    for b in range(2):
        o, nch = (o0, n0) if b == 0 else (o1, n1)

        @pl.when(o > 0)
        def _():
            flush(b, nch)
        cntbuf[pl.ds(0, L)] = jnp.full((L,), nch * CH + o, jnp.int32)
        pltpu.sync_copy(cntbuf,
                        cnt_hbm.at[pl.ds(((2 * c + b) * NS + s) * L, L)])


_bucket = pl.kernel(
    _bucket_body,
    out_type=[
        jax.ShapeDtypeStruct((Q + 64,), jnp.int32),
        jax.ShapeDtypeStruct((Q + 64,), jnp.float32),
        jax.ShapeDtypeStruct((Q + 64,), jnp.int32),
        jax.ShapeDtypeStruct((4 * NS * L,), jnp.int32),
    ],
    mesh=_MESH,
    scratch_types=[
        pltpu.VMEM((B,), jnp.int32),
        pltpu.VMEM((B,), jnp.int32),
        pltpu.VMEM((B,), jnp.float32),
        pltpu.VMEM((CH + B,), jnp.int32),
        pltpu.VMEM((CH + B,), jnp.float32),
        pltpu.VMEM((CH + B,), jnp.int32),
        pltpu.VMEM((CH + B,), jnp.int32),
        pltpu.VMEM((CH + B,), jnp.float32),
        pltpu.VMEM((CH + B,), jnp.int32),
        pltpu.VMEM((L,), jnp.int32),
    ],
    compiler_params=_PARAMS,
)


def _hop_body(ego_hbm, colq_hbm, valq_hbm, idxq_hbm, cnt_hbm, out_hbm,
              colb0, valb0, idxb0, colb1, valb1, idxb1,
              gbuf0, gbuf1, zbuf, cntv, acc, semE, semG, semS):
    c = lax.axis_index("c")
    s = lax.axis_index("s")
    lanes = lax.iota(jnp.int32, L)
    ebufs = ((colb0, valb0, idxb0), (colb1, valb1, idxb1))
    gbufs = (gbuf0, gbuf1)

    zeros = jnp.zeros((L,), jnp.float32)
    for i in range(ZROWS):
        for j in range(D // L):
            zbuf[i, pl.ds(j * L, L)] = zeros

    for p in range(2):
        base = (2 * c + p) * P
        qbase = ((2 * c + p) * NS + s) * QTILE

        for i in range(ROWS_TILE // ZROWS):
            pltpu.sync_copy(zbuf, acc.at[pl.ds(s * ROWS_TILE + i * ZROWS, ZROWS)])
        plsc.subcore_barrier()

        pltpu.sync_copy(cnt_hbm.at[pl.ds(((2 * c + p) * NS + s) * L, L)], cntv)
        cnt = cntv[pl.ds(0, L)]
        nb = (cnt[0] + (B - 1)) // B

        def fire_edges(bi, buf):
            colb, valb, idxb = ebufs[buf]
            qb = qbase + bi * B
            pltpu.async_copy(colq_hbm.at[pl.ds(qb, B)], colb, semE)
            pltpu.async_copy(valq_hbm.at[pl.ds(qb, B)], valb, semE)
            pltpu.async_copy(idxq_hbm.at[pl.ds(qb, B)], idxb, semE)

        def wait_edges(buf):
            colb, valb, idxb = ebufs[buf]
            pltpu.make_async_copy(colq_hbm.at[pl.ds(0, B)], colb, semE).wait()
            pltpu.make_async_copy(valq_hbm.at[pl.ds(0, B)], valb, semE).wait()
            pltpu.make_async_copy(idxq_hbm.at[pl.ds(0, B)], idxb, semE).wait()

        def sanitize(bi, buf):
            colb, _, idxb = ebufs[buf]
            for g in range(B // L):
                gpos = lanes + (bi * B + g * L)
                ok = gpos < cnt
                cl = colb[pl.ds(g * L, L)]
                colb[pl.ds(g * L, L)] = jnp.where(ok, cl, 0)
                li = idxb[pl.ds(g * L, L)]
                idxb[pl.ds(g * L, L)] = jnp.where(ok, li, DUMMY)

        def fire_gather(buf):
            pltpu.async_copy(ego_hbm.at[ebufs[buf][0]], gbufs[buf], semG)

        def wait_gather(buf):
            pltpu.make_async_copy(ego_hbm.at[ebufs[buf][0]], gbufs[buf],
                                  semG).wait()

        def fire_scatter(buf):
            pltpu.async_copy(gbufs[buf], acc.at[ebufs[buf][2]], semS, add=True)

        def wait_scatter(buf):
            pltpu.make_async_copy(gbufs[buf], acc.at[ebufs[buf][2]],
                                  semS).wait()

        @pl.when(nb > 0)
        def _():
            fire_edges(0, 0)
            wait_edges(0)
            sanitize(0, 0)
            fire_gather(0)

        def step(i, cur, nxt):
            live = i < nb

            @pl.when((i >= 1) & live)
            def _():
                wait_scatter(nxt)

            @pl.when(i + 1 < nb)
            def _():
                fire_edges(i + 1, nxt)

            @pl.when(live)
            def _():
                wait_gather(cur)

            @pl.when(i + 1 < nb)
            def _():
                wait_edges(nxt)
                sanitize(i + 1, nxt)
                fire_gather(nxt)

            @pl.when(live)
            def _():
                gbuf = gbufs[cur]
                valbuf = ebufs[cur][1]
                for g in range(B // L):
                    vals_g = valbuf[pl.ds(g * L, L)]
                    for jj in range(L):
                        j = g * L + jj
                        vj = vals_g.at[jnp.full((L,), jj, jnp.int32)].get(
                            mode="promise_in_bounds")
                        for f in range(D // L):
                            gbuf[j, pl.ds(f * L, L)] = (
                                gbuf[j, pl.ds(f * L, L)] * vj)
                fire_scatter(cur)

        def pair_body(t, carry):
            step(2 * t, 0, 1)
            step(2 * t + 1, 1, 0)
            return carry

        lax.fori_loop(0, (nb + 1) // 2, pair_body, 0)

        @pl.when(nb > 0)
        def _():
            wait_scatter_last = pltpu.make_async_copy(
                gbuf0, acc.at[idxb0], semS)
            wait_scatter_last.wait()
        plsc.subcore_barrier()
        pltpu.sync_copy(acc.at[pl.ds(s * ROWS_TILE, ROWS_TILE)],
                        out_hbm.at[pl.ds(base + s * ROWS_TILE, ROWS_TILE)])
        plsc.subcore_barrier()


_hop = pl.kernel(
    _hop_body,
    out_type=jax.ShapeDtypeStruct((N_PAD, D), jnp.float32),
    mesh=_MESH,
    scratch_types=[
        pltpu.VMEM((B,), jnp.int32),
        pltpu.VMEM((B,), jnp.float32),
        pltpu.VMEM((B,), jnp.int32),
        pltpu.VMEM((B,), jnp.int32),
        pltpu.VMEM((B,), jnp.float32),
        pltpu.VMEM((B,), jnp.int32),
        pltpu.VMEM((B, D), jnp.float32),
        pltpu.VMEM((B, D), jnp.float32),
        pltpu.VMEM((ZROWS, D), jnp.float32),
        pltpu.VMEM((L,), jnp.int32),
        pltpu.VMEM_SHARED((P + 8, D), jnp.float32),
        pltpu.SemaphoreType.DMA,
        pltpu.SemaphoreType.DMA,
        pltpu.SemaphoreType.DMA,
    ],
    compiler_params=_PARAMS,
)


def kernel(user_embed, item_embed, adj_row, adj_col, adj_val):
    ego0 = jnp.concatenate([user_embed, item_embed], axis=0)
    ego0 = jnp.pad(ego0, ((0, N_PAD - N_NODES), (0, 0)))
    row = jnp.pad(adj_row.astype(jnp.int32), (0, E_PAD - E),
                  constant_values=jnp.int32(1 << 20))
    col = jnp.pad(adj_col.astype(jnp.int32), (0, E_PAD - E))
    val = jnp.pad(adj_val, (0, E_PAD - E))

    colq, valq, idxq, cnts = _bucket(row, col, val)
    e1 = _hop(ego0, colq, valq, idxq, cnts)
    e2 = _hop(e1, colq, valq, idxq, cnts)
    e3 = _hop(e2, colq, valq, idxq, cnts)

    mean = (e1 + e2 + e3) * jnp.float32(1.0 / 3.0)
    user_all = mean[:N_USERS]
    item_all = mean[N_USERS:N_NODES]
    user_layer = e1[:N_USERS]
    item_layer = e1[N_USERS:N_NODES]
    return (user_all, item_all, user_layer, item_layer)


# speedup vs baseline: 291.0858x; 1.3958x over previous
import jax
import jax.numpy as jnp
from jax import lax
from jax.experimental import pallas as pl
from jax.experimental.pallas import tpu as pltpu
from jax.experimental.pallas import tpu_sc as plsc

N_USERS = 50000
N_NODES = 100000
D = 64
E = 1250000

NC = 2
NS = 16
L = 16

P = 25600
N_PAD = 4 * P
B = 128
E_PAD = ((E + NS * B - 1) // (NS * B)) * (NS * B)
TE = E_PAD // NS
ROWS_TILE = P // NS
ZROWS = 160

SCC = 6016
NSC = TE // SCC
KR = SCC // B
CH = 1024
QTILE = ((TE + CH - 1) // CH) * CH
Q = 4 * NS * QTILE
TRASH = Q
DUMMY = P

_MESH = plsc.VectorSubcoreMesh(core_axis_name="c", subcore_axis_name="s",
                               num_cores=NC, num_subcores=NS)
_PARAMS = pltpu.CompilerParams(use_tc_tiling_on_sc=False)


def _bucket_body(row_hbm, col_hbm, val_hbm,
                 colq_hbm, valq_hbm, idxq_hbm, cnt_hbm,
                 rowb0, colb0, valb0, rowb1, colb1, valb1,
                 qcol0, qval0, qidx0, qcol1, qval1, qidx1, cntbuf, semE):
    c = lax.axis_index("c")
    s = lax.axis_index("s")
    lanes = lax.iota(jnp.int32, L)
    ebufs = ((rowb0, colb0, valb0), (rowb1, colb1, valb1))

    def psum16(mi):
        p = mi
        for sh in (1, 2, 4, 8):
            idx = jnp.maximum(lanes - sh, 0)
            shifted = p.at[idx].get(mode="promise_in_bounds")
            p = p + jnp.where(lanes >= sh, shifted, jnp.int32(0))
        return p

    qbufs = ((qcol0, qval0, qidx0), (qcol1, qval1, qidx1))

    def flush(b, nch):
        qcol, qval, qidx = qbufs[b]
        qbase = ((2 * c + b) * NS + s) * QTILE + nch * CH
        pltpu.sync_copy(qcol.at[pl.ds(0, CH)], colq_hbm.at[pl.ds(qbase, CH)])
        pltpu.sync_copy(qval.at[pl.ds(0, CH)], valq_hbm.at[pl.ds(qbase, CH)])
        pltpu.sync_copy(qidx.at[pl.ds(0, CH)], idxq_hbm.at[pl.ds(qbase, CH)])

    def fire_edges(bi, buf):
        rowb, colb, valb = ebufs[buf]
        eb = s * TE + bi * B
        pltpu.async_copy(row_hbm.at[pl.ds(eb, B)], rowb, semE)
        pltpu.async_copy(col_hbm.at[pl.ds(eb, B)], colb, semE)
        pltpu.async_copy(val_hbm.at[pl.ds(eb, B)], valb, semE)

    def wait_edges(buf):
        rowb, colb, valb = ebufs[buf]
        pltpu.make_async_copy(row_hbm.at[pl.ds(0, B)], rowb, semE).wait()
        pltpu.make_async_copy(col_hbm.at[pl.ds(0, B)], colb, semE).wait()
        pltpu.make_async_copy(val_hbm.at[pl.ds(0, B)], valb, semE).wait()

    NBATCH = TE // B

    def step(i, cur, o0, o1, n0, n1):
        @pl.when(i + 1 < NBATCH)
        def _():
            fire_edges(i + 1, 1 - cur)
        wait_edges(cur)
        rowbuf, colbuf, valbuf = ebufs[cur]
        for j in range(B // L):
            r = rowbuf[pl.ds(j * L, L)]
            cl = colbuf[pl.ds(j * L, L)]
            vv = valbuf[pl.ds(j * L, L)]
            for b in range(2):
                qcol, qval, qidx = qbufs[b]
                o = o0 if b == 0 else o1
                lr = r - (2 * c + b) * P
                m = (lr >= 0) & (lr < P)
                mi = jnp.where(m, jnp.int32(1), jnp.int32(0))
                pos = psum16(mi)
                posm = jnp.where(m, pos, jnp.int32(0))
                srcv = lanes
                for l in range(L):
                    pm = posm.at[jnp.full((L,), l, jnp.int32)].get(
                        mode="promise_in_bounds")
                    srcv = jnp.where(lanes + 1 == pm, jnp.int32(l), srcv)
                qcol[pl.ds(o, L)] = cl.at[srcv].get(mode="promise_in_bounds")
                qval[pl.ds(o, L)] = vv.at[srcv].get(mode="promise_in_bounds")
                lrd = jnp.where(m, lr, DUMMY)
                qidx[pl.ds(o, L)] = lrd.at[srcv].get(mode="promise_in_bounds")
                o = o + pos[L - 1]
                if b == 0:
                    o0 = o
                else:
                    o1 = o
        for b in range(2):
            o, nch = (o0, n0) if b == 0 else (o1, n1)
            qcol, qval, qidx = qbufs[b]

            @pl.when(o >= CH)
            def _():
                flush(b, nch)
                for k in range(B // L):
                    rc = qcol[pl.ds(CH + k * L, L)]
                    rv = qval[pl.ds(CH + k * L, L)]
                    ri = qidx[pl.ds(CH + k * L, L)]
                    qcol[pl.ds(k * L, L)] = rc
                    qval[pl.ds(k * L, L)] = rv
                    qidx[pl.ds(k * L, L)] = ri

            nch = nch + jnp.where(o >= CH, jnp.int32(1), jnp.int32(0))
            o = jnp.where(o >= CH, o - CH, o)
            if b == 0:
                o0, n0 = o, nch
            else:
                o1, n1 = o, nch
        return (o0, o1, n0, n1)

    def pair_body(t, carry):
        o0, o1, n0, n1 = step(2 * t, 0, *carry)
        return step(2 * t + 1, 1, o0, o1, n0, n1)

    fire_edges(0, 0)
    z = jnp.int32(0)
    o0, o1, n0, n1 = lax.fori_loop(0, NBATCH // 2, pair_body, (z, z, z, z))
    if NBATCH % 2:
        o0, o1, n0, n1 = step(NBATCH - 1, 0, o0, o1, n0, n1)

    for b in range(2):
        o, nch = (o0, n0) if b == 0 else (o1, n1)

        @pl.when(o > 0)
        def _():
            flush(b, nch)
        cntbuf[pl.ds(0, L)] = jnp.full((L,), nch * CH + o, jnp.int32)
        pltpu.sync_copy(cntbuf,
                        cnt_hbm.at[pl.ds(((2 * c + b) * NS + s) * L, L)])


_bucket = pl.kernel(
    _bucket_body,
    out_type=[
        jax.ShapeDtypeStruct((Q + 64,), jnp.int32),
        jax.ShapeDtypeStruct((Q + 64,), jnp.float32),
        jax.ShapeDtypeStruct((Q + 64,), jnp.int32),
        jax.ShapeDtypeStruct((4 * NS * L,), jnp.int32),
    ],
    mesh=_MESH,
    scratch_types=[
        pltpu.VMEM((B,), jnp.int32),
        pltpu.VMEM((B,), jnp.int32),
        pltpu.VMEM((B,), jnp.float32),
        pltpu.VMEM((B,), jnp.int32),
        pltpu.VMEM((B,), jnp.int32),
        pltpu.VMEM((B,), jnp.float32),
        pltpu.VMEM((CH + B,), jnp.int32),
        pltpu.VMEM((CH + B,), jnp.float32),
        pltpu.VMEM((CH + B,), jnp.int32),
        pltpu.VMEM((CH + B,), jnp.int32),
        pltpu.VMEM((CH + B,), jnp.float32),
        pltpu.VMEM((CH + B,), jnp.int32),
        pltpu.VMEM((L,), jnp.int32),
        pltpu.SemaphoreType.DMA,
    ],
    compiler_params=_PARAMS,
)


def _hop_body(ego_hbm, colq_hbm, valq_hbm, idxq_hbm, cnt_hbm, out_hbm,
              colb0, valb0, idxb0, colb1, valb1, idxb1,
              gbuf0, gbuf1, zbuf, cntv, acc, semE, semG, semS):
    c = lax.axis_index("c")
    s = lax.axis_index("s")
    lanes = lax.iota(jnp.int32, L)
    ebufs = ((colb0, valb0, idxb0), (colb1, valb1, idxb1))
    gbufs = (gbuf0, gbuf1)

    zeros = jnp.zeros((L,), jnp.float32)
    for i in range(ZROWS):
        for j in range(D // L):
            zbuf[i, pl.ds(j * L, L)] = zeros

    for p in range(2):
        base = (2 * c + p) * P
        qbase = ((2 * c + p) * NS + s) * QTILE

        for i in range(ROWS_TILE // ZROWS):
            pltpu.sync_copy(zbuf, acc.at[pl.ds(s * ROWS_TILE + i * ZROWS, ZROWS)])
        plsc.subcore_barrier()

        pltpu.sync_copy(cnt_hbm.at[pl.ds(((2 * c + p) * NS + s) * L, L)], cntv)
        cnt = cntv[pl.ds(0, L)]
        nb = (cnt[0] + (B - 1)) // B

        def fire_edges(bi, buf):
            colb, valb, idxb = ebufs[buf]
            qb = qbase + bi * B
            pltpu.async_copy(colq_hbm.at[pl.ds(qb, B)], colb, semE)
            pltpu.async_copy(valq_hbm.at[pl.ds(qb, B)], valb, semE)
            pltpu.async_copy(idxq_hbm.at[pl.ds(qb, B)], idxb, semE)

        def wait_edges(buf):
            colb, valb, idxb = ebufs[buf]
            pltpu.make_async_copy(colq_hbm.at[pl.ds(0, B)], colb, semE).wait()
            pltpu.make_async_copy(valq_hbm.at[pl.ds(0, B)], valb, semE).wait()
            pltpu.make_async_copy(idxq_hbm.at[pl.ds(0, B)], idxb, semE).wait()

        def sanitize(bi, buf):
            colb, _, idxb = ebufs[buf]
            for g in range(B // L):
                gpos = lanes + (bi * B + g * L)
                ok = gpos < cnt
                cl = colb[pl.ds(g * L, L)]
                colb[pl.ds(g * L, L)] = jnp.where(ok, cl, 0)
                li = idxb[pl.ds(g * L, L)]
                idxb[pl.ds(g * L, L)] = jnp.where(ok, li, DUMMY)

        def fire_gather(buf):
            pltpu.async_copy(ego_hbm.at[ebufs[buf][0]], gbufs[buf], semG)

        def wait_gather(buf):
            pltpu.make_async_copy(ego_hbm.at[ebufs[buf][0]], gbufs[buf],
                                  semG).wait()

        def fire_scatter(buf):
            pltpu.async_copy(gbufs[buf], acc.at[ebufs[buf][2]], semS, add=True)

        def wait_scatter(buf):
            pltpu.make_async_copy(gbufs[buf], acc.at[ebufs[buf][2]],
                                  semS).wait()

        @pl.when(nb > 0)
        def _():
            fire_edges(0, 0)
            wait_edges(0)
            sanitize(0, 0)
            fire_gather(0)

        def step(i, cur, nxt):
            live = i < nb

            @pl.when((i >= 1) & live)
            def _():
                wait_scatter(nxt)

            @pl.when(i + 1 < nb)
            def _():
                fire_edges(i + 1, nxt)

            @pl.when(live)
            def _():
                wait_gather(cur)

            @pl.when(i + 1 < nb)
            def _():
                wait_edges(nxt)
                sanitize(i + 1, nxt)
                fire_gather(nxt)

            @pl.when(live)
            def _():
                gbuf = gbufs[cur]
                valbuf = ebufs[cur][1]
                for g in range(B // L):
                    vals_g = valbuf[pl.ds(g * L, L)]
                    for jj in range(L):
                        j = g * L + jj
                        vj = vals_g.at[jnp.full((L,), jj, jnp.int32)].get(
                            mode="promise_in_bounds")
                        for f in range(D // L):
                            gbuf[j, pl.ds(f * L, L)] = (
                                gbuf[j, pl.ds(f * L, L)] * vj)
                fire_scatter(cur)

        def pair_body(t, carry):
            step(2 * t, 0, 1)
            step(2 * t + 1, 1, 0)
            return carry

        lax.fori_loop(0, (nb + 1) // 2, pair_body, 0)

        @pl.when(nb > 0)
        def _():
            wait_scatter_last = pltpu.make_async_copy(
                gbuf0, acc.at[idxb0], semS)
            wait_scatter_last.wait()
        plsc.subcore_barrier()
        pltpu.sync_copy(acc.at[pl.ds(s * ROWS_TILE, ROWS_TILE)],
                        out_hbm.at[pl.ds(base + s * ROWS_TILE, ROWS_TILE)])
        plsc.subcore_barrier()


_hop = pl.kernel(
    _hop_body,
    out_type=jax.ShapeDtypeStruct((N_PAD, D), jnp.float32),
    mesh=_MESH,
    scratch_types=[
        pltpu.VMEM((B,), jnp.int32),
        pltpu.VMEM((B,), jnp.float32),
        pltpu.VMEM((B,), jnp.int32),
        pltpu.VMEM((B,), jnp.int32),
        pltpu.VMEM((B,), jnp.float32),
        pltpu.VMEM((B,), jnp.int32),
        pltpu.VMEM((B, D), jnp.float32),
        pltpu.VMEM((B, D), jnp.float32),
        pltpu.VMEM((ZROWS, D), jnp.float32),
        pltpu.VMEM((L,), jnp.int32),
        pltpu.VMEM_SHARED((P + 8, D), jnp.float32),
        pltpu.SemaphoreType.DMA,
        pltpu.SemaphoreType.DMA,
        pltpu.SemaphoreType.DMA,
    ],
    compiler_params=_PARAMS,
)


def kernel(user_embed, item_embed, adj_row, adj_col, adj_val):
    ego0 = jnp.concatenate([user_embed, item_embed], axis=0)
    ego0 = jnp.pad(ego0, ((0, N_PAD - N_NODES), (0, 0)))
    row = jnp.pad(adj_row.astype(jnp.int32), (0, E_PAD - E),
                  constant_values=jnp.int32(1 << 20))
    col = jnp.pad(adj_col.astype(jnp.int32), (0, E_PAD - E))
    val = jnp.pad(adj_val, (0, E_PAD - E))

    colq, valq, idxq, cnts = _bucket(row, col, val)
    e1 = _hop(ego0, colq, valq, idxq, cnts)
    e2 = _hop(e1, colq, valq, idxq, cnts)
    e3 = _hop(e2, colq, valq, idxq, cnts)

    mean = (e1 + e2 + e3) * jnp.float32(1.0 / 3.0)
    user_all = mean[:N_USERS]
    item_all = mean[N_USERS:N_NODES]
    user_layer = e1[:N_USERS]
    item_layer = e1[N_USERS:N_NODES]
    return (user_all, item_all, user_layer, item_layer)
